# initial kernel scaffold (unmeasured)
def kernel(x, router_W, route_idx, expert_W):
    import jax
    import jax.numpy as jnp
    from jax import lax
    from jax.experimental import pallas as pl
    from jax.experimental.pallas import tpu as pltpu

    N_DEV = 32
    N_EXP = 128
    EPG = 4
    CAP = 204
    M, D = x.shape
    H = expert_W.shape[-1]

    def body(x_ref, ridx_ref, ew_ref, out_ref, wbuf, histbuf,
             wsend, wrecv, hsend, hrecv):
        my = lax.axis_index("i")
        left = lax.rem(my + N_DEV - 1, N_DEV)
        right = lax.rem(my + 1, N_DEV)

        barrier_sem = pltpu.get_barrier_semaphore()
        pl.semaphore_signal(barrier_sem, inc=1, device_id=(left,),
                            device_id_type=pl.DeviceIdType.MESH)
        pl.semaphore_signal(barrier_sem, inc=1, device_id=(right,),
                            device_id_type=pl.DeviceIdType.MESH)
        pl.semaphore_wait(barrier_sem, 2)

        xbf = x_ref[...].astype(jnp.bfloat16)
        ridx = ridx_ref[...]

        wbuf[my] = ew_ref[...].astype(jnp.bfloat16)

        e_iota = lax.broadcasted_iota(jnp.int32, (M, N_EXP), 1)
        oh = (ridx == e_iota).astype(jnp.float32)
        histbuf[my, :] = jnp.sum(oh, axis=0).astype(jnp.int32)

        def compute_block(origin):
            parts = []
            for sub in range(EPG):
                e = origin * EPG + sub
                m = ridx == e
                parts.append(jnp.where(m, xbf, jnp.zeros_like(xbf)))
            xm = jnp.concatenate(parts, axis=1)
            w = wbuf[origin].reshape(EPG * D, H)
            return jnp.dot(xm, w, preferred_element_type=jnp.float32)

        out_ref[...] = compute_block(my)

        for h in range(N_DEV - 1):
            send_idx = lax.rem(my - h + 2 * N_DEV, N_DEV)
            w_rdma = pltpu.make_async_remote_copy(
                src_ref=wbuf.at[send_idx],
                dst_ref=wbuf.at[send_idx],
                send_sem=wsend.at[h],
                recv_sem=wrecv.at[h],
                device_id=(right,),
                device_id_type=pl.DeviceIdType.MESH,
            )
            h_rdma = pltpu.make_async_remote_copy(
                src_ref=histbuf.at[send_idx],
                dst_ref=histbuf.at[send_idx],
                send_sem=hsend.at[h],
                recv_sem=hrecv.at[h],
                device_id=(right,),
                device_id_type=pl.DeviceIdType.MESH,
            )
            w_rdma.start()
            h_rdma.start()
            w_rdma.wait()
            h_rdma.wait()
            recv_idx = lax.rem(my - h - 1 + 2 * N_DEV, N_DEV)
            out_ref[...] += compute_block(recv_idx)

        t_iota = lax.broadcasted_iota(jnp.int32, (N_DEV, N_EXP), 0)
        prior = jnp.where(t_iota < my, histbuf[...], 0).astype(jnp.float32)
        off = jnp.sum(prior, axis=0)
        off_tok = jnp.sum(oh * off[None, :], axis=1, keepdims=True)
        eq = ridx == ridx.reshape(1, M)
        i_row = lax.broadcasted_iota(jnp.int32, (M, M), 0)
        i_col = lax.broadcasted_iota(jnp.int32, (M, M), 1)
        rank = jnp.sum(
            jnp.logical_and(eq, i_col < i_row).astype(jnp.float32),
            axis=1, keepdims=True,
        )
        accept = (off_tok + rank) < CAP
        out_ref[...] = jnp.where(accept, out_ref[...], 0.0)

    out_shape = jax.ShapeDtypeStruct((M, H), jnp.float32)
    return pl.pallas_call(
        body,
        out_shape=out_shape,
        in_specs=[
            pl.BlockSpec(memory_space=pltpu.VMEM),
            pl.BlockSpec(memory_space=pltpu.VMEM),
            pl.BlockSpec(memory_space=pltpu.VMEM),
        ],
        out_specs=pl.BlockSpec(memory_space=pltpu.VMEM),
        scratch_shapes=[
            pltpu.VMEM((N_DEV, EPG, D, H), jnp.bfloat16),
            pltpu.VMEM((N_DEV, N_EXP), jnp.int32),
            pltpu.SemaphoreType.DMA((N_DEV - 1,)),
            pltpu.SemaphoreType.DMA((N_DEV - 1,)),
            pltpu.SemaphoreType.DMA((N_DEV - 1,)),
            pltpu.SemaphoreType.DMA((N_DEV - 1,)),
        ],
        compiler_params=pltpu.CompilerParams(collective_id=0),
    )(x, route_idx, expert_W)


# baseline (device time: 454083 ns/iter reference)
def kernel(x, router_W, route_idx, expert_W):
    import jax
    import jax.numpy as jnp
    from jax import lax
    from jax.experimental import pallas as pl
    from jax.experimental.pallas import tpu as pltpu

    N_DEV = 32
    N_EXP = 128
    EPG = 4
    CAP = 204
    M, D = x.shape
    H = expert_W.shape[-1]

    def body(x_ref, ridx_ref, ew_ref, out_ref, wbuf, histbuf,
             wsend, wrecv, hsend, hrecv):
        my = lax.axis_index("i")
        left = lax.rem(my + N_DEV - 1, N_DEV)
        right = lax.rem(my + 1, N_DEV)

        barrier_sem = pltpu.get_barrier_semaphore()
        pl.semaphore_signal(barrier_sem, inc=1, device_id=(left,),
                            device_id_type=pl.DeviceIdType.MESH)
        pl.semaphore_signal(barrier_sem, inc=1, device_id=(right,),
                            device_id_type=pl.DeviceIdType.MESH)
        pl.semaphore_wait(barrier_sem, 2)

        xbf = x_ref[...].astype(jnp.bfloat16)
        ridx = ridx_ref[...]

        wbuf[my] = ew_ref[...].astype(jnp.bfloat16)

        e_iota = lax.broadcasted_iota(jnp.int32, (M, N_EXP), 1)
        oh = (ridx == e_iota).astype(jnp.float32)
        histbuf[my, :] = jnp.sum(oh, axis=0).astype(jnp.int32)

        def compute_block(origin):
            parts = []
            for sub in range(EPG):
                e = origin * EPG + sub
                m = ridx == e
                parts.append(jnp.where(m, xbf, jnp.zeros_like(xbf)))
            xm = jnp.concatenate(parts, axis=1)
            w = wbuf[origin].reshape(EPG * D, H)
            return jnp.dot(xm, w, preferred_element_type=jnp.float32)

        out_ref[...] = compute_block(my)

        for h in range(N_DEV - 1):
            send_idx = lax.rem(my - h + 2 * N_DEV, N_DEV)
            w_rdma = pltpu.make_async_remote_copy(
                src_ref=wbuf.at[send_idx],
                dst_ref=wbuf.at[send_idx],
                send_sem=wsend.at[h],
                recv_sem=wrecv.at[h],
                device_id=(right,),
                device_id_type=pl.DeviceIdType.MESH,
            )
            h_rdma = pltpu.make_async_remote_copy(
                src_ref=histbuf.at[send_idx],
                dst_ref=histbuf.at[send_idx],
                send_sem=hsend.at[h],
                recv_sem=hrecv.at[h],
                device_id=(right,),
                device_id_type=pl.DeviceIdType.MESH,
            )
            w_rdma.start()
            h_rdma.start()
            w_rdma.wait()
            h_rdma.wait()
            recv_idx = lax.rem(my - h - 1 + 2 * N_DEV, N_DEV)
            out_ref[...] += compute_block(recv_idx)

        t_iota = lax.broadcasted_iota(jnp.int32, (N_DEV, N_EXP), 0)
        prior = jnp.where(t_iota < my, histbuf[...], 0).astype(jnp.float32)
        off = jnp.sum(prior, axis=0)
        off_tok = jnp.sum(oh * off[None, :], axis=1, keepdims=True)
        eq = ridx == ridx.reshape(1, M)
        i_row = lax.broadcasted_iota(jnp.int32, (M, M), 0)
        i_col = lax.broadcasted_iota(jnp.int32, (M, M), 1)
        rank = jnp.sum(
            jnp.logical_and(eq, i_col < i_row).astype(jnp.float32),
            axis=1, keepdims=True,
        )
        accept = (off_tok + rank) < CAP
        out_ref[...] = jnp.where(accept, out_ref[...], 0.0)

    out_shape = jax.ShapeDtypeStruct((M, H), jnp.float32)
    return pl.pallas_call(
        body,
        out_shape=out_shape,
        in_specs=[
            pl.BlockSpec(memory_space=pltpu.VMEM),
            pl.BlockSpec(memory_space=pltpu.VMEM),
            pl.BlockSpec(memory_space=pltpu.VMEM),
        ],
        out_specs=pl.BlockSpec(memory_space=pltpu.VMEM),
        scratch_shapes=[
            pltpu.VMEM((N_DEV, EPG, D, H), jnp.bfloat16),
            pltpu.VMEM((N_DEV, N_EXP), jnp.int32),
            pltpu.SemaphoreType.DMA((N_DEV - 1,)),
            pltpu.SemaphoreType.DMA((N_DEV - 1,)),
            pltpu.SemaphoreType.DMA((N_DEV - 1,)),
            pltpu.SemaphoreType.DMA((N_DEV - 1,)),
        ],
        compiler_params=pltpu.CompilerParams(
            collective_id=0,
            vmem_limit_bytes=100 * 1024 * 1024,
        ),
    )(x, route_idx, expert_W)


# device time: 407058 ns/iter; 1.1155x vs baseline; 1.1155x over previous
def kernel(x, router_W, route_idx, expert_W):
    import jax
    import jax.numpy as jnp
    from jax import lax
    from jax.experimental import pallas as pl
    from jax.experimental.pallas import tpu as pltpu

    N_DEV = 32
    N_EXP = 128
    EPG = 4
    CAP = 204
    M, D = x.shape
    H = expert_W.shape[-1]
    R_HOPS = 16
    L_HOPS = 15

    def body(x_ref, ridx_ref, ew_ref, out_ref, wbuf, histbuf,
             rs, rr, ls, lr, hrs, hrr, hls, hlr):
        my = lax.axis_index("i")
        left = lax.rem(my + N_DEV - 1, N_DEV)
        right = lax.rem(my + 1, N_DEV)

        def slot(k):
            return lax.rem(k + 2 * N_DEV, N_DEV)

        barrier_sem = pltpu.get_barrier_semaphore()
        pl.semaphore_signal(barrier_sem, inc=1, device_id=(left,),
                            device_id_type=pl.DeviceIdType.MESH)
        pl.semaphore_signal(barrier_sem, inc=1, device_id=(right,),
                            device_id_type=pl.DeviceIdType.MESH)
        pl.semaphore_wait(barrier_sem, 2)

        xbf = x_ref[...].astype(jnp.bfloat16)
        ridx = ridx_ref[...]

        wbuf[my] = ew_ref[...].astype(jnp.bfloat16)

        e_iota = lax.broadcasted_iota(jnp.int32, (M, N_EXP), 1)
        oh = (ridx == e_iota).astype(jnp.float32)
        histbuf[my, :] = jnp.sum(oh, axis=0).astype(jnp.int32)

        def mk(buf, idx, ssem, rsem, dev):
            return pltpu.make_async_remote_copy(
                src_ref=buf.at[idx],
                dst_ref=buf.at[idx],
                send_sem=ssem,
                recv_sem=rsem,
                device_id=(dev,),
                device_id_type=pl.DeviceIdType.MESH,
            )

        r_rd = [mk(wbuf, slot(my - h), rs.at[h], rr.at[h], right)
                for h in range(R_HOPS)]
        l_rd = [mk(wbuf, slot(my + h), ls.at[h], lr.at[h], left)
                for h in range(L_HOPS)]
        hr_rd = [mk(histbuf, slot(my - h), hrs.at[h], hrr.at[h], right)
                 for h in range(R_HOPS)]
        hl_rd = [mk(histbuf, slot(my + h), hls.at[h], hlr.at[h], left)
                 for h in range(L_HOPS)]

        def compute_block(origin):
            parts = []
            for sub in range(EPG):
                e = origin * EPG + sub
                m = ridx == e
                parts.append(jnp.where(m, xbf, jnp.zeros_like(xbf)))
            xm = jnp.concatenate(parts, axis=1)
            w = wbuf[origin].reshape(EPG * D, H)
            return jnp.dot(xm, w, preferred_element_type=jnp.float32)

        r_rd[0].start()
        l_rd[0].start()
        hr_rd[0].start()
        hl_rd[0].start()

        out_ref[...] = compute_block(my)

        eq = ridx == ridx.reshape(1, M)
        i_row = lax.broadcasted_iota(jnp.int32, (M, M), 0)
        i_col = lax.broadcasted_iota(jnp.int32, (M, M), 1)
        rank = jnp.sum(
            jnp.logical_and(eq, i_col < i_row).astype(jnp.float32),
            axis=1, keepdims=True,
        )

        for h in range(R_HOPS):
            has_l = h < L_HOPS
            r_rd[h].wait_recv()
            hr_rd[h].wait_recv()
            if has_l:
                l_rd[h].wait_recv()
                hl_rd[h].wait_recv()
            if h + 1 < R_HOPS:
                r_rd[h + 1].start()
                hr_rd[h + 1].start()
            if h + 1 < L_HOPS:
                l_rd[h + 1].start()
                hl_rd[h + 1].start()
            out_ref[...] += compute_block(slot(my - 1 - h))
            if has_l:
                out_ref[...] += compute_block(slot(my + 1 + h))

        t_iota = lax.broadcasted_iota(jnp.int32, (N_DEV, N_EXP), 0)
        prior = jnp.where(t_iota < my, histbuf[...], 0).astype(jnp.float32)
        off = jnp.sum(prior, axis=0)
        off_tok = jnp.sum(oh * off[None, :], axis=1, keepdims=True)
        accept = (off_tok + rank) < CAP
        out_ref[...] = jnp.where(accept, out_ref[...], 0.0)

        for rd in r_rd + l_rd + hr_rd + hl_rd:
            rd.wait_send()

    out_shape = jax.ShapeDtypeStruct((M, H), jnp.float32)
    return pl.pallas_call(
        body,
        out_shape=out_shape,
        in_specs=[
            pl.BlockSpec(memory_space=pltpu.VMEM),
            pl.BlockSpec(memory_space=pltpu.VMEM),
            pl.BlockSpec(memory_space=pltpu.VMEM),
        ],
        out_specs=pl.BlockSpec(memory_space=pltpu.VMEM),
        scratch_shapes=[
            pltpu.VMEM((N_DEV, EPG, D, H), jnp.bfloat16),
            pltpu.VMEM((N_DEV, N_EXP), jnp.int32),
            pltpu.SemaphoreType.DMA((R_HOPS,)),
            pltpu.SemaphoreType.DMA((R_HOPS,)),
            pltpu.SemaphoreType.DMA((L_HOPS,)),
            pltpu.SemaphoreType.DMA((L_HOPS,)),
            pltpu.SemaphoreType.DMA((R_HOPS,)),
            pltpu.SemaphoreType.DMA((R_HOPS,)),
            pltpu.SemaphoreType.DMA((L_HOPS,)),
            pltpu.SemaphoreType.DMA((L_HOPS,)),
        ],
        compiler_params=pltpu.CompilerParams(
            collective_id=0,
            vmem_limit_bytes=100 * 1024 * 1024,
        ),
    )(x, route_idx, expert_W)


# device time: 123640 ns/iter; 3.6726x vs baseline; 3.2923x over previous
def kernel(x, router_W, route_idx, expert_W):
    import jax
    import jax.numpy as jnp
    from jax import lax
    from jax.experimental import pallas as pl
    from jax.experimental.pallas import tpu as pltpu

    N_DEV = 32
    N_EXP = 128
    EPG = 4
    CAP = 204
    K = 40
    M, D = x.shape
    H = expert_W.shape[-1]
    S = N_DEV * EPG * K

    def body(x_ref, ridx_ref, ew_ref, out_ref,
             sbuf, rbuf, ybuf, rres, hbuf,
             hsend, dsend, ysend, hrecv, drecv, yrecv):
        my = lax.axis_index("i")

        barrier_sem = pltpu.get_barrier_semaphore()
        for r in range(N_DEV - 1):
            peer = lax.rem(my + 1 + r, N_DEV)
            pl.semaphore_signal(barrier_sem, inc=1, device_id=(peer,),
                                device_id_type=pl.DeviceIdType.MESH)
        pl.semaphore_wait(barrier_sem, N_DEV - 1)

        def mk(src, dst, ssem, rsem, dev):
            return pltpu.make_async_remote_copy(
                src_ref=src, dst_ref=dst, send_sem=ssem, recv_sem=rsem,
                device_id=(dev,), device_id_type=pl.DeviceIdType.MESH,
            )

        xbf = x_ref[...].astype(jnp.bfloat16)
        ridx = ridx_ref[...]

        e_iota = lax.broadcasted_iota(jnp.int32, (M, N_EXP), 1)
        oh = (ridx == e_iota).astype(jnp.float32)
        hist = jnp.sum(oh, axis=0).astype(jnp.int32)
        hbuf[N_DEV - 1, :] = hist

        h_rd = [mk(hbuf.at[N_DEV - 1], hbuf.at[N_DEV - 2 - r],
                   hsend.at[r], hrecv.at[0], lax.rem(my + 1 + r, N_DEV))
                for r in range(N_DEV - 1)]
        for rd in h_rd:
            rd.start()

        eq = ridx == ridx.reshape(1, M)
        i_row = lax.broadcasted_iota(jnp.int32, (M, M), 0)
        i_col = lax.broadcasted_iota(jnp.int32, (M, M), 1)
        rank = jnp.sum(
            jnp.logical_and(eq, i_col < i_row).astype(jnp.int32),
            axis=1, keepdims=True,
        )

        owner = lax.div(ridx, EPG)
        sub = lax.rem(ridx, EPG)
        rel = lax.rem(owner - my - 1 + 2 * N_DEV, N_DEV)
        slot = rel * (EPG * K) + sub * K + rank
        slot = jnp.where(rank < K, slot, S)

        d_iota = lax.broadcasted_iota(jnp.int32, (S, M), 0)
        disp = (d_iota == slot.reshape(1, M)).astype(jnp.bfloat16)
        sflat = jnp.dot(disp, xbf, preferred_element_type=jnp.float32)
        sbuf[...] = sflat.astype(jnp.bfloat16).reshape(N_DEV, EPG, K, D)
        rbuf[N_DEV - 1] = sbuf[N_DEV - 1]

        d_rd = [mk(sbuf.at[r], rbuf.at[N_DEV - 2 - r],
                   dsend.at[r], drecv.at[0], lax.rem(my + 1 + r, N_DEV))
                for r in range(N_DEV - 1)]
        for rd in d_rd:
            rd.start()

        mk(hbuf.at[pl.ds(0, N_DEV - 1)], hbuf.at[pl.ds(0, N_DEV - 1)],
           hrecv.at[0], hrecv.at[0], my).wait_recv()
        r_iota = lax.broadcasted_iota(jnp.int32, (N_DEV - 1, N_EXP), 0)
        prior = jnp.where(r_iota >= N_DEV - 1 - my,
                          hbuf[pl.ds(0, N_DEV - 1), :], 0)
        off = jnp.sum(prior.astype(jnp.float32), axis=0)
        off_tok = jnp.sum(oh * off[None, :], axis=1, keepdims=True)
        accept = (off_tok + rank.astype(jnp.float32)) < CAP

        mk(rbuf.at[pl.ds(0, N_DEV - 1)], rbuf.at[pl.ds(0, N_DEV - 1)],
           drecv.at[0], drecv.at[0], my).wait_recv()
        wbf = ew_ref[...].astype(jnp.bfloat16)
        for s in range(EPG):
            toks = rbuf[:, s].reshape(N_DEV * K, D)
            y = jnp.dot(toks, wbf[s], preferred_element_type=jnp.float32)
            ybuf[:, s] = y.astype(jnp.bfloat16).reshape(N_DEV, K, H)
        rres[N_DEV - 1] = ybuf[N_DEV - 1]

        y_rd = [mk(ybuf.at[r], rres.at[N_DEV - 2 - r],
                   ysend.at[r], yrecv.at[0], lax.rem(my + 1 + r, N_DEV))
                for r in range(N_DEV - 1)]
        for rd in y_rd:
            rd.start()

        mk(rres.at[pl.ds(0, N_DEV - 1)], rres.at[pl.ds(0, N_DEV - 1)],
           yrecv.at[0], yrecv.at[0], my).wait_recv()
        c_iota = lax.broadcasted_iota(jnp.int32, (M, S), 1)
        comb = (c_iota == slot).astype(jnp.bfloat16)
        res = jnp.dot(comb, rres[...].reshape(S, H),
                      preferred_element_type=jnp.float32)
        out_ref[...] = jnp.where(accept, res, 0.0)

        for rd in h_rd + d_rd + y_rd:
            rd.wait_send()

    out_shape = jax.ShapeDtypeStruct((M, H), jnp.float32)
    return pl.pallas_call(
        body,
        out_shape=out_shape,
        in_specs=[
            pl.BlockSpec(memory_space=pltpu.VMEM),
            pl.BlockSpec(memory_space=pltpu.VMEM),
            pl.BlockSpec(memory_space=pltpu.VMEM),
        ],
        out_specs=pl.BlockSpec(memory_space=pltpu.VMEM),
        scratch_shapes=[
            pltpu.VMEM((N_DEV, EPG, K, D), jnp.bfloat16),
            pltpu.VMEM((N_DEV, EPG, K, D), jnp.bfloat16),
            pltpu.VMEM((N_DEV, EPG, K, H), jnp.bfloat16),
            pltpu.VMEM((N_DEV, EPG, K, H), jnp.bfloat16),
            pltpu.VMEM((N_DEV, N_EXP), jnp.int32),
            pltpu.SemaphoreType.DMA((N_DEV - 1,)),
            pltpu.SemaphoreType.DMA((N_DEV - 1,)),
            pltpu.SemaphoreType.DMA((N_DEV - 1,)),
            pltpu.SemaphoreType.DMA((1,)),
            pltpu.SemaphoreType.DMA((1,)),
            pltpu.SemaphoreType.DMA((1,)),
        ],
        compiler_params=pltpu.CompilerParams(
            collective_id=0,
            vmem_limit_bytes=100 * 1024 * 1024,
        ),
    )(x, route_idx, expert_W)
